# Initial kernel scaffold; baseline (speedup 1.0000x reference)
import jax
import jax.numpy as jnp
from jax.experimental import pallas as pl


def kernel(x):
    raise NotImplementedError("write your pallas kernel here")


# speedup vs baseline: 26.6978x; 26.6978x over previous
import jax
import jax.numpy as jnp
from jax.experimental import pallas as pl
from jax.experimental.pallas import tpu as pltpu

N = 8192
K = N - 1 - N // 2


def _body(x_ref, o_ref):
    xb = x_ref[...]
    u = jax.lax.bitcast_convert_type(xb, jnp.uint32)
    mask = jnp.where((u >> 31) == 1, jnp.uint32(0xFFFFFFFF), jnp.uint32(0x80000000))
    key = u ^ mask

    shape = (xb.shape[0], xb.shape[1], 1)
    p0 = jnp.zeros(shape, jnp.uint32)

    def step(t, p):
        bit = jax.lax.shift_left(jnp.uint32(1), jnp.uint32(31) - t.astype(jnp.uint32))
        cand = p | bit
        cnt = jnp.sum((key < cand).astype(jnp.int32), axis=2, keepdims=True)
        return jnp.where(cnt <= K, cand, p)

    p = jax.lax.fori_loop(0, 32, step, p0)

    pos = (p >> 31) == 1
    bits = jnp.where(pos, p ^ jnp.uint32(0x80000000), ~p)
    o_ref[...] = jax.lax.bitcast_convert_type(bits, jnp.float32)[:, :, 0]


def kernel(x):
    B0, B1, n = x.shape
    TB = 8
    return pl.pallas_call(
        _body,
        grid=(B0 // TB,),
        in_specs=[pl.BlockSpec((TB, B1, n), lambda i: (i, 0, 0))],
        out_specs=pl.BlockSpec((TB, B1), lambda i: (i, 0)),
        out_shape=jax.ShapeDtypeStruct((B0, B1), x.dtype),
    )(x)
